# baseline (speedup 1.0000x reference)
import functools

import jax
import jax.numpy as jnp
from jax import lax
from jax.experimental import pallas as pl
from jax.experimental.pallas import tpu as pltpu
from jax.experimental.pallas import tpu_sc as plsc

NUM_WORKERS = 32
L = 16
CP = 128
WT = 129


@functools.partial(jax.jit, static_argnums=(2, 3, 4, 5))
def _sc_embed(idx, table, B, K, H, W):
    D = table.shape[1]
    N = H * W
    bpw = B // NUM_WORKERS
    cpb = N // CP
    T = bpw * cpb
    rows_per_chunk = CP // W
    scale = 1.0 / K

    mesh = plsc.VectorSubcoreMesh(core_axis_name="c", subcore_axis_name="s")

    @functools.partial(
        pl.kernel,
        out_type=jax.ShapeDtypeStruct((B, D, H, W), jnp.float32),
        mesh=mesh,
        scratch_types=[
            pltpu.VMEM((bpw, K, N), jnp.int32),
            pltpu.VMEM((2, CP, D), jnp.float32),
            pltpu.VMEM((2, D, WT), jnp.float32),
            pltpu.SemaphoreType.DMA,
            pltpu.SemaphoreType.DMA,
            pltpu.SemaphoreType.DMA,
            pltpu.SemaphoreType.DMA,
            pltpu.SemaphoreType.DMA,
            pltpu.SemaphoreType.DMA,
        ],
        compiler_params=pltpu.CompilerParams(use_tc_tiling_on_sc=False,
                                             needs_layout_passes=False),
    )
    def body(idx_hbm, table_hbm, out_hbm, idx_v, acc, acc_t,
             si0, si1, sg0, sg1, so0, so1):
        wid = lax.axis_index("s") * 2 + lax.axis_index("c")
        iota = lax.iota(jnp.int32, L)
        zeros = jnp.zeros((L,), jnp.float32)
        sem_idx, sem_g, sem_out = (si0, si1), (sg0, sg1), (so0, so1)

        def fire_idx(t):
            b = wid * bpw + t
            pltpu.async_copy(idx_hbm.at[b], idx_v.at[t], sem_idx[t])

        def wait_idx(t):
            b = wid * bpw + t
            pltpu.make_async_copy(idx_hbm.at[b], idx_v.at[t],
                                  sem_idx[t]).wait()

        def zero_acc(j):
            for p in range(CP):
                for half in range(D // L):
                    acc[j, p, pl.ds(half * L, L)] = zeros

        def fire_gathers(c, j):
            t = c // cpb
            base = (c % cpb) * CP
            for k in range(K):
                pltpu.async_copy(
                    table_hbm.at[idx_v.at[t, k, pl.ds(base, CP)]],
                    acc.at[j], sem_g[j], add=True)

        def wait_gathers(c, j):
            t = c // cpb
            base = (c % cpb) * CP
            for k in range(K):
                pltpu.make_async_copy(
                    table_hbm.at[idx_v.at[t, k, pl.ds(base, CP)]],
                    acc.at[j], sem_g[j]).wait()

        def out_slices(c, j):
            b = wid * bpw + (c // cpb)
            h0 = (c % cpb) * rows_per_chunk
            return [(acc_t.at[j, :, pl.ds(r * W, W)],
                     out_hbm.at[b, :, h0 + r, :])
                    for r in range(rows_per_chunk)]

        def fire_out(c, j):
            for src, dst in out_slices(c, j):
                pltpu.async_copy(src, dst, sem_out[j])

        def wait_out(c, j):
            for src, dst in out_slices(c, j):
                pltpu.make_async_copy(src, dst, sem_out[j]).wait()

        def transpose_chunk(j):
            jvec = jnp.full((L,), j, jnp.int32)
            dvecs = [iota + half * L for half in range(D // L)]
            for p in range(CP):
                pvec = jnp.full((L,), p, jnp.int32)
                for half in range(D // L):
                    val = acc[j, p, pl.ds(half * L, L)] * scale
                    plsc.store_scatter(acc_t, [jvec, dvecs[half], pvec], val)

        def half_iter(i2, j):
            c = 2 * i2 + j
            j2 = 1 - j

            if bpw > 1 and j == (cpb - 1) % 2:
                pl.when(c == cpb - 1)(lambda: wait_idx(1))

            def prep_next():
                zero_acc(j2)
                fire_gathers(c + 1, j2)
            if j == 0:
                prep_next()
            else:
                pl.when(i2 < T // 2 - 1)(prep_next)

            wait_gathers(c, j)
            pl.when(i2 >= 1)(lambda: wait_out(c - 2, j))
            transpose_chunk(j)
            fire_out(c, j)

        def step(i2, _):
            half_iter(i2, 0)
            half_iter(i2, 1)
            return 0

        for t in range(bpw):
            fire_idx(t)
        wait_idx(0)
        zero_acc(0)
        fire_gathers(0, 0)

        lax.fori_loop(0, T // 2, step, 0, unroll=False)

        wait_out(T - 2, 0)
        wait_out(T - 1, 1)

    return body(idx, table)


def kernel(inputs, table):
    B, K, H, W = inputs.shape
    idx = inputs.reshape(B, K, H * W).astype(jnp.int32)
    return _sc_embed(idx, table, B, K, H, W)


# speedup vs baseline: 1.0644x; 1.0644x over previous
import functools

import jax
import jax.numpy as jnp
from jax import lax
from jax.experimental import pallas as pl
from jax.experimental.pallas import tpu as pltpu
from jax.experimental.pallas import tpu_sc as plsc

NUM_WORKERS = 32
L = 16


WP = 65


@functools.partial(jax.jit, static_argnums=(2, 3, 4, 5))
def _sc_embed(idx, table, B, K, H, W):
    D = table.shape[1]
    bpw = B // NUM_WORKERS
    T = bpw * H
    scale = 1.0 / K

    mesh = plsc.VectorSubcoreMesh(core_axis_name="c", subcore_axis_name="s")

    @functools.partial(
        pl.kernel,
        out_type=jax.ShapeDtypeStruct((B, D, H, W), jnp.float32),
        mesh=mesh,
        scratch_types=[
            pltpu.VMEM((2, K, W), jnp.int32),
            pltpu.VMEM((2, W, D), jnp.bfloat16),
            pltpu.VMEM((2, D, WP), jnp.float32),
            pltpu.SemaphoreType.DMA,
            pltpu.SemaphoreType.DMA,
            pltpu.SemaphoreType.DMA,
            pltpu.SemaphoreType.DMA,
            pltpu.SemaphoreType.DMA,
            pltpu.SemaphoreType.DMA,
        ],
        compiler_params=pltpu.CompilerParams(use_tc_tiling_on_sc=False,
                                             needs_layout_passes=False),
    )
    def body(idx_hbm, table_hbm, out_hbm, idx_v, acc, acc_t,
             si0, si1, sg0, sg1, so0, so1):
        wid = lax.axis_index("s") * 2 + lax.axis_index("c")
        iota = lax.iota(jnp.int32, L)
        zeros = jnp.zeros((2 * L,), jnp.bfloat16)
        sem_idx, sem_g, sem_out = (si0, si1), (sg0, sg1), (so0, so1)

        def bh(c):
            return wid * bpw + (c // H), c % H

        def fire_idx(c, j):
            b, h = bh(c)
            pltpu.async_copy(idx_hbm.at[b, :, h, :], idx_v.at[j], sem_idx[j])

        def wait_idx(c, j):
            b, h = bh(c)
            pltpu.make_async_copy(
                idx_hbm.at[b, :, h, :], idx_v.at[j], sem_idx[j]).wait()

        def zero_acc(j):
            for p in range(W):
                for grp in range(D // (2 * L)):
                    acc[j, p, pl.ds(grp * 2 * L, 2 * L)] = zeros

        def fire_gathers(j):
            for k in range(K):
                pltpu.async_copy(table_hbm.at[idx_v.at[j, k]], acc.at[j],
                                 sem_g[j], add=True)

        def wait_gathers(j):
            for k in range(K):
                pltpu.make_async_copy(table_hbm.at[idx_v.at[j, k]],
                                      acc.at[j], sem_g[j]).wait()

        def fire_out(c, j):
            b, h = bh(c)
            pltpu.async_copy(acc_t.at[j, :, pl.ds(0, W)],
                             out_hbm.at[b, :, h, :], sem_out[j])

        def wait_out(c, j):
            b, h = bh(c)
            pltpu.make_async_copy(acc_t.at[j, :, pl.ds(0, W)],
                                  out_hbm.at[b, :, h, :], sem_out[j]).wait()

        def transpose_chunk(j):
            jvec = jnp.full((L,), j, jnp.int32)
            dvecs = [2 * iota + grp * 2 * L + sub
                     for grp in range(D // (2 * L)) for sub in range(2)]
            for p in range(W):
                pvec = jnp.full((L,), p, jnp.int32)
                for grp in range(D // (2 * L)):
                    v32 = acc[j, p, pl.ds(grp * 2 * L, 2 * L)]
                    lo, hi = plsc.unpack(v32, format=plsc.PackFormat.INTERLEAVED)
                    for sub, v in enumerate((lo, hi)):
                        plsc.store_scatter(
                            acc_t, [jvec, dvecs[2 * grp + sub], pvec],
                            v.astype(jnp.float32) * scale)

        def half_iter(i2, j):
            c = 2 * i2 + j
            j2 = 1 - j

            def prep_next():
                wait_idx(c + 1, j2)
                zero_acc(j2)
                fire_gathers(j2)
            if j == 0:
                prep_next()
            else:
                pl.when(i2 < T // 2 - 1)(prep_next)

            wait_gathers(j)
            pl.when(c + 2 < T)(lambda: fire_idx(c + 2, j))
            pl.when(i2 >= 1)(lambda: wait_out(c - 2, j))
            transpose_chunk(j)
            fire_out(c, j)

        def step(i2, _):
            half_iter(i2, 0)
            half_iter(i2, 1)
            return 0

        fire_idx(0, 0)
        fire_idx(1, 1)
        wait_idx(0, 0)
        zero_acc(0)
        fire_gathers(0)

        lax.fori_loop(0, T // 2, step, 0, unroll=False)

        wait_out(T - 2, 0)
        wait_out(T - 1, 1)

    return body(idx, table)


def kernel(inputs, table):
    B, K, H, W = inputs.shape
    idx = inputs.astype(jnp.int32)
    return _sc_embed(idx, table.astype(jnp.bfloat16), B, K, H, W)


# speedup vs baseline: 1.2740x; 1.1969x over previous
import functools

import jax
import jax.numpy as jnp
from jax import lax
from jax.experimental import pallas as pl
from jax.experimental.pallas import tpu as pltpu
from jax.experimental.pallas import tpu_sc as plsc

NUM_WORKERS = 32
L = 16


WP = 65


@functools.partial(jax.jit, static_argnums=(2, 3, 4, 5))
def _sc_embed(idx, table, B, K, H, W):
    D = table.shape[1]
    bpw = B // NUM_WORKERS
    T = bpw * H
    scale = 1.0 / K

    mesh = plsc.VectorSubcoreMesh(core_axis_name="c", subcore_axis_name="s")

    @functools.partial(
        pl.kernel,
        out_type=jax.ShapeDtypeStruct((B, D, H, W), jnp.float32),
        mesh=mesh,
        scratch_types=[
            pltpu.VMEM((2, K, W), jnp.int32),
            pltpu.VMEM((2, W, D), jnp.bfloat16),
            pltpu.VMEM((2, D, WP), jnp.float32),
            pltpu.VMEM_SHARED((100000, 32), jnp.bfloat16),
            pltpu.SemaphoreType.DMA,
            pltpu.SemaphoreType.DMA,
            pltpu.SemaphoreType.DMA,
            pltpu.SemaphoreType.DMA,
            pltpu.SemaphoreType.DMA,
            pltpu.SemaphoreType.DMA,
            pltpu.SemaphoreType.DMA,
        ],
        compiler_params=pltpu.CompilerParams(use_tc_tiling_on_sc=False,
                                             needs_layout_passes=False),
    )
    def body(idx_hbm, table_hbm, out_hbm, idx_v, acc, acc_t, table_s, st,
             si0, si1, sg0, sg1, so0, so1):
        wid = lax.axis_index("s") * 2 + lax.axis_index("c")
        iota = lax.iota(jnp.int32, L)
        zeros = jnp.zeros((2 * L,), jnp.bfloat16)
        sem_idx, sem_g, sem_out = (si0, si1), (sg0, sg1), (so0, so1)

        def bh(c):
            return wid * bpw + (c // H), c % H

        def fire_idx(c, j):
            b, h = bh(c)
            pltpu.async_copy(idx_hbm.at[b, :, h, :], idx_v.at[j], sem_idx[j])

        def wait_idx(c, j):
            b, h = bh(c)
            pltpu.make_async_copy(
                idx_hbm.at[b, :, h, :], idx_v.at[j], sem_idx[j]).wait()

        def zero_acc(j):
            for p in range(W):
                for grp in range(D // (2 * L)):
                    acc[j, p, pl.ds(grp * 2 * L, 2 * L)] = zeros

        def fire_gathers(j):
            for k in range(K):
                pltpu.async_copy(table_s.at[idx_v.at[j, k]], acc.at[j],
                                 sem_g[j], add=True)

        def wait_gathers(j):
            for k in range(K):
                pltpu.make_async_copy(table_s.at[idx_v.at[j, k]],
                                      acc.at[j], sem_g[j]).wait()

        def fire_out(c, j):
            b, h = bh(c)
            pltpu.async_copy(acc_t.at[j, :, pl.ds(0, W)],
                             out_hbm.at[b, :, h, :], sem_out[j])

        def wait_out(c, j):
            b, h = bh(c)
            pltpu.make_async_copy(acc_t.at[j, :, pl.ds(0, W)],
                                  out_hbm.at[b, :, h, :], sem_out[j]).wait()

        def transpose_chunk(j):
            jvec = jnp.full((L,), j, jnp.int32)
            dvecs = [2 * iota + grp * 2 * L + sub
                     for grp in range(D // (2 * L)) for sub in range(2)]
            for p in range(W):
                pvec = jnp.full((L,), p, jnp.int32)
                for grp in range(D // (2 * L)):
                    v32 = acc[j, p, pl.ds(grp * 2 * L, 2 * L)]
                    lo, hi = plsc.unpack(v32, format=plsc.PackFormat.INTERLEAVED)
                    for sub, v in enumerate((lo, hi)):
                        plsc.store_scatter(
                            acc_t, [jvec, dvecs[2 * grp + sub], pvec],
                            v.astype(jnp.float32) * scale)

        def half_iter(i2, j):
            c = 2 * i2 + j
            j2 = 1 - j

            def prep_next():
                wait_idx(c + 1, j2)
                zero_acc(j2)
                fire_gathers(j2)
            if j == 0:
                prep_next()
            else:
                pl.when(i2 < T // 2 - 1)(prep_next)

            wait_gathers(j)
            pl.when(c + 2 < T)(lambda: fire_idx(c + 2, j))
            pl.when(i2 >= 1)(lambda: wait_out(c - 2, j))
            transpose_chunk(j)
            fire_out(c, j)

        def step(i2, _):
            half_iter(i2, 0)
            half_iter(i2, 1)
            return 0

        v_per_tile = table.shape[0] // 16
        start = lax.axis_index("s") * v_per_tile
        pltpu.async_copy(table_hbm.at[pl.ds(start, v_per_tile)],
                         table_s.at[pl.ds(start, v_per_tile)], st).wait()
        plsc.subcore_barrier()

        fire_idx(0, 0)
        fire_idx(1, 1)
        wait_idx(0, 0)
        zero_acc(0)
        fire_gathers(0)

        lax.fori_loop(0, T // 2, step, 0, unroll=False)

        wait_out(T - 2, 0)
        wait_out(T - 1, 1)

    return body(idx, table)


def kernel(inputs, table):
    B, K, H, W = inputs.shape
    idx = inputs.astype(jnp.int32)
    return _sc_embed(idx, table.astype(jnp.bfloat16), B, K, H, W)


# speedup vs baseline: 1.2772x; 1.0025x over previous
import functools

import jax
import jax.numpy as jnp
from jax import lax
from jax.experimental import pallas as pl
from jax.experimental.pallas import tpu as pltpu
from jax.experimental.pallas import tpu_sc as plsc

NUM_WORKERS = 32
L = 16


WP = 65


@functools.partial(jax.jit, static_argnums=(2, 3, 4, 5))
def _sc_embed(idx, table, B, K, H, W):
    D = table.shape[1]
    bpw = B // NUM_WORKERS
    T = bpw * H
    scale = 1.0 / K

    mesh = plsc.VectorSubcoreMesh(core_axis_name="c", subcore_axis_name="s")

    @functools.partial(
        pl.kernel,
        out_type=jax.ShapeDtypeStruct((B, D, H, W), jnp.float32),
        mesh=mesh,
        scratch_types=[
            pltpu.VMEM((2, K, W), jnp.int32),
            pltpu.VMEM((2, W, D), jnp.bfloat16),
            pltpu.VMEM((2, D, WP), jnp.float32),
            pltpu.VMEM_SHARED(table.shape, jnp.bfloat16),
            pltpu.SemaphoreType.DMA,
            pltpu.SemaphoreType.DMA,
            pltpu.SemaphoreType.DMA,
            pltpu.SemaphoreType.DMA,
            pltpu.SemaphoreType.DMA,
            pltpu.SemaphoreType.DMA,
            pltpu.SemaphoreType.DMA,
        ],
        compiler_params=pltpu.CompilerParams(use_tc_tiling_on_sc=False,
                                             needs_layout_passes=False),
    )
    def body(idx_hbm, table_hbm, out_hbm, idx_v, acc, acc_t, table_s, st,
             si0, si1, sg0, sg1, so0, so1):
        wid = lax.axis_index("s") * 2 + lax.axis_index("c")
        iota = lax.iota(jnp.int32, L)
        zeros = jnp.zeros((2 * L,), jnp.bfloat16)
        sem_idx, sem_g, sem_out = (si0, si1), (sg0, sg1), (so0, so1)

        def bh(c):
            return wid * bpw + (c // H), c % H

        def fire_idx(c, j):
            b, h = bh(c)
            pltpu.async_copy(idx_hbm.at[b, :, h, :], idx_v.at[j], sem_idx[j])

        def wait_idx(c, j):
            b, h = bh(c)
            pltpu.make_async_copy(
                idx_hbm.at[b, :, h, :], idx_v.at[j], sem_idx[j]).wait()

        def zero_acc(j):
            for p in range(W):
                for grp in range(D // (2 * L)):
                    acc[j, p, pl.ds(grp * 2 * L, 2 * L)] = zeros

        def fire_gathers(j):
            for k in range(K):
                pltpu.async_copy(table_s.at[idx_v.at[j, k]], acc.at[j],
                                 sem_g[j], add=True)

        def wait_gathers(j):
            for k in range(K):
                pltpu.make_async_copy(table_s.at[idx_v.at[j, k]],
                                      acc.at[j], sem_g[j]).wait()

        def fire_out(c, j):
            b, h = bh(c)
            pltpu.async_copy(acc_t.at[j, :, pl.ds(0, W)],
                             out_hbm.at[b, :, h, :], sem_out[j])

        def wait_out(c, j):
            b, h = bh(c)
            pltpu.make_async_copy(acc_t.at[j, :, pl.ds(0, W)],
                                  out_hbm.at[b, :, h, :], sem_out[j]).wait()

        def transpose_chunk(j):
            jvec = jnp.full((L,), j, jnp.int32)
            dvecs = [2 * iota + grp * 2 * L + sub
                     for grp in range(D // (2 * L)) for sub in range(2)]
            for p in range(W):
                pvec = jnp.full((L,), p, jnp.int32)
                for grp in range(D // (2 * L)):
                    v32 = acc[j, p, pl.ds(grp * 2 * L, 2 * L)]
                    lo, hi = plsc.unpack(v32, format=plsc.PackFormat.INTERLEAVED)
                    for sub, v in enumerate((lo, hi)):
                        plsc.store_scatter(
                            acc_t, [jvec, dvecs[2 * grp + sub], pvec],
                            v.astype(jnp.float32) * scale)

        def half_iter(i2, j):
            c = 2 * i2 + j
            j2 = 1 - j

            def prep_next():
                wait_idx(c + 1, j2)
                zero_acc(j2)
                fire_gathers(j2)
            if j == 0:
                prep_next()
            else:
                pl.when(i2 < T // 2 - 1)(prep_next)

            wait_gathers(j)
            pl.when(c + 2 < T)(lambda: fire_idx(c + 2, j))
            pl.when(i2 >= 1)(lambda: wait_out(c - 2, j))
            transpose_chunk(j)
            fire_out(c, j)

        def step(i2, _):
            half_iter(i2, 0)
            half_iter(i2, 1)
            return 0

        v_per_tile = table.shape[0] // 16
        start = lax.axis_index("s") * v_per_tile
        pltpu.async_copy(table_hbm.at[pl.ds(start, v_per_tile)],
                         table_s.at[pl.ds(start, v_per_tile)], st).wait()
        plsc.subcore_barrier()

        fire_idx(0, 0)
        fire_idx(1, 1)
        wait_idx(0, 0)
        zero_acc(0)
        fire_gathers(0)

        lax.fori_loop(0, T // 2, step, 0, unroll=False)

        wait_out(T - 2, 0)
        wait_out(T - 1, 1)

    return body(idx, table)


def kernel(inputs, table):
    B, K, H, W = inputs.shape
    idx = inputs.astype(jnp.int32)
    return _sc_embed(idx, table.astype(jnp.bfloat16), B, K, H, W)
